# Initial kernel scaffold; baseline (speedup 1.0000x reference)
import jax
import jax.numpy as jnp
from jax.experimental import pallas as pl


def kernel(x, edge_index, W0, b0, W1, b1, W2, b2, g0, be0, g1, be1, g2, be2, Wc1, bc1, Wc2, bc2):
    raise NotImplementedError("write your pallas kernel here")


# speedup vs baseline: 12.0740x; 12.0740x over previous
import functools

import jax
import jax.numpy as jnp
from jax import lax
from jax.experimental import pallas as pl
from jax.experimental.pallas import tpu as pltpu
from jax.experimental.pallas import tpu_sc as plsc

N = 10000
E = 320000
F_IN = 128
H = 64
C = 10

NC = 2
NS = 16
NW = NC * NS
EPW = E // NW
CH = 80
NCHUNK = EPW // CH
TPT = 640
ZR = 80

_MESH = plsc.VectorSubcoreMesh(core_axis_name="c", subcore_axis_name="s")
_SC_PARAMS = pltpu.CompilerParams(use_tc_tiling_on_sc=False)


@functools.partial(
    pl.kernel,
    out_type=jax.ShapeDtypeStruct((NC * N, 8), jnp.float32),
    mesh=_MESH,
    compiler_params=_SC_PARAMS,
    scratch_types=[
        pltpu.VMEM((CH,), jnp.int32),
        pltpu.VMEM((CH, 8), jnp.float32),
        pltpu.VMEM((ZR, 8), jnp.float32),
        pltpu.VMEM_SHARED((N, 8), jnp.float32),
    ],
)
def _sc_degree(col_hbm, ones_hbm, zeros_hbm, out_hbm, cidx, ones_v, zbuf, acc):
    c = lax.axis_index("c")
    s = lax.axis_index("s")
    wid = s * NC + c
    nj = jnp.where(s == NS - 1, (N - (NS - 1) * TPT) // ZR, TPT // ZR)

    pltpu.sync_copy(ones_hbm, ones_v)
    pltpu.sync_copy(zeros_hbm, zbuf)

    def zero_body(j, _):
        r0 = pl.multiple_of(s * TPT + j * ZR, 8)
        pltpu.sync_copy(zbuf, acc.at[pl.ds(r0, ZR)])
        return _

    lax.fori_loop(0, nj, zero_body, None)
    plsc.subcore_barrier()

    def body(k, _):
        base = pl.multiple_of(wid * EPW + k * CH, 8)
        pltpu.sync_copy(col_hbm.at[pl.ds(base, CH)], cidx)
        pltpu.sync_copy(ones_v, acc.at[cidx], add=True)
        return _

    lax.fori_loop(0, NCHUNK, body, None)
    plsc.subcore_barrier()

    def out_body(j, _):
        r0 = pl.multiple_of(s * TPT + j * ZR, 8)
        pltpu.sync_copy(acc.at[pl.ds(r0, ZR)], zbuf)
        pltpu.sync_copy(zbuf, out_hbm.at[pl.ds(c * N + r0, ZR)])
        return _

    lax.fori_loop(0, nj, out_body, None)


@functools.partial(
    pl.kernel,
    out_type=jax.ShapeDtypeStruct((NC * N, H), jnp.float32),
    mesh=_MESH,
    compiler_params=_SC_PARAMS,
    scratch_types=[
        pltpu.VMEM((CH,), jnp.int32),
        pltpu.VMEM((CH,), jnp.int32),
        pltpu.VMEM((CH, H), jnp.float32),
        pltpu.VMEM((ZR, H), jnp.float32),
        pltpu.VMEM_SHARED((N, H), jnp.float32),
        pltpu.SemaphoreType.DMA,
    ],
)
def _sc_aggregate(y_hbm, row_hbm, col_hbm, zeros_hbm, out_hbm,
                  ridx, cidx, buf, zbuf, acc, sem):
    c = lax.axis_index("c")
    s = lax.axis_index("s")
    wid = s * NC + c
    nj = jnp.where(s == NS - 1, (N - (NS - 1) * TPT) // ZR, TPT // ZR)

    pltpu.sync_copy(zeros_hbm, zbuf)

    def zero_body(j, _):
        r0 = pl.multiple_of(s * TPT + j * ZR, 8)
        pltpu.sync_copy(zbuf, acc.at[pl.ds(r0, ZR)])
        return _

    lax.fori_loop(0, nj, zero_body, None)
    plsc.subcore_barrier()

    def body(k, _):
        base = pl.multiple_of(wid * EPW + k * CH, 8)
        pltpu.sync_copy(row_hbm.at[pl.ds(base, CH)], ridx)
        pltpu.sync_copy(col_hbm.at[pl.ds(base, CH)], cidx)
        pltpu.async_copy(y_hbm.at[ridx], buf, sem).wait()
        pltpu.sync_copy(buf, acc.at[cidx], add=True)
        return _

    lax.fori_loop(0, NCHUNK, body, None)
    plsc.subcore_barrier()

    def out_body(j, _):
        r0 = pl.multiple_of(s * TPT + j * ZR, 8)
        pltpu.sync_copy(acc.at[pl.ds(r0, ZR)], zbuf)
        pltpu.sync_copy(zbuf, out_hbm.at[pl.ds(c * N + r0, ZR)])
        return _

    lax.fori_loop(0, nj, out_body, None)


def _tc_prep_body(deg_ref, x_ref, w0_ref, y_ref, xw_ref, dinv_ref):
    d8 = deg_ref[...]
    deg = d8[:N, 0:1] + d8[N:, 0:1] + 1.0
    dinv = lax.rsqrt(deg)
    xw = jnp.dot(x_ref[...], w0_ref[...], preferred_element_type=jnp.float32)
    y_ref[...] = xw * dinv
    xw_ref[...] = xw
    dinv_ref[...] = dinv


def _tc_prep(deg_parts, x, W0):
    return pl.pallas_call(
        _tc_prep_body,
        out_shape=[
            jax.ShapeDtypeStruct((N, H), jnp.float32),
            jax.ShapeDtypeStruct((N, H), jnp.float32),
            jax.ShapeDtypeStruct((N, 1), jnp.float32),
        ],
    )(deg_parts, x, W0)


def _layer_out(sp, xw, dinv, b, g, be):
    s = sp[:N] + sp[N:]
    o = dinv * s + (dinv * dinv) * xw + b
    mu = jnp.mean(o, axis=0, keepdims=True)
    var = jnp.mean((o - mu) * (o - mu), axis=0, keepdims=True)
    return jnp.maximum((o - mu) * lax.rsqrt(var + 1e-5) * g + be, 0.0)


def _tc_post_body(sp_ref, xw_ref, dinv_ref, b_ref, g_ref, be_ref, wn_ref,
                  yn_ref, xwn_ref):
    dinv = dinv_ref[...]
    h = _layer_out(sp_ref[...], xw_ref[...], dinv, b_ref[...], g_ref[...],
                   be_ref[...])
    xwn = jnp.dot(h, wn_ref[...], preferred_element_type=jnp.float32)
    yn_ref[...] = xwn * dinv
    xwn_ref[...] = xwn


def _tc_post(s_parts, xw, dinv, b, g, be, Wn):
    return pl.pallas_call(
        _tc_post_body,
        out_shape=[
            jax.ShapeDtypeStruct((N, H), jnp.float32),
            jax.ShapeDtypeStruct((N, H), jnp.float32),
        ],
    )(s_parts, xw, dinv, b.reshape(1, H), g.reshape(1, H), be.reshape(1, H),
      Wn)


def _tc_final_body(sp_ref, xw_ref, dinv_ref, b_ref, g_ref, be_ref,
                   wc1_ref, bc1_ref, wc2_ref, bc2_ref, out_ref):
    h = _layer_out(sp_ref[...], xw_ref[...], dinv_ref[...], b_ref[...],
                   g_ref[...], be_ref[...])
    hc = jnp.maximum(
        jnp.dot(h, wc1_ref[...], preferred_element_type=jnp.float32)
        + bc1_ref[...], 0.0)
    out_ref[...] = (
        jnp.dot(hc, wc2_ref[...], preferred_element_type=jnp.float32)
        + bc2_ref[...])


def _tc_final(s_parts, xw, dinv, b, g, be, Wc1, bc1, Wc2, bc2):
    return pl.pallas_call(
        _tc_final_body,
        out_shape=jax.ShapeDtypeStruct((N, C), jnp.float32),
    )(s_parts, xw, dinv, b.reshape(1, H), g.reshape(1, H), be.reshape(1, H),
      Wc1, bc1.reshape(1, H // 2), Wc2, bc2.reshape(1, C))


def kernel(x, edge_index, W0, b0, W1, b1, W2, b2, g0, be0, g1, be1, g2, be2,
           Wc1, bc1, Wc2, bc2):
    row = edge_index[0]
    col = edge_index[1]
    ones8 = jnp.ones((CH, 8), jnp.float32)
    zeros8 = jnp.zeros((ZR, 8), jnp.float32)
    zerosH = jnp.zeros((ZR, H), jnp.float32)

    deg_parts = _sc_degree(col, ones8, zeros8)
    y0, xw0, dinv = _tc_prep(deg_parts, x, W0)
    s0 = _sc_aggregate(y0, row, col, zerosH)
    y1, xw1 = _tc_post(s0, xw0, dinv, b0, g0, be0, W1)
    s1 = _sc_aggregate(y1, row, col, zerosH)
    y2, xw2 = _tc_post(s1, xw1, dinv, b1, g1, be1, W2)
    s2 = _sc_aggregate(y2, row, col, zerosH)
    return _tc_final(s2, xw2, dinv, b2, g2, be2, Wc1, bc1, Wc2, bc2)


# speedup vs baseline: 31.6345x; 2.6201x over previous
import functools

import jax
import jax.numpy as jnp
from jax import lax
from jax.experimental import pallas as pl
from jax.experimental.pallas import tpu as pltpu
from jax.experimental.pallas import tpu_sc as plsc

N = 10000
E = 320000
F_IN = 128
H = 64
C = 10

NC = 2
NS = 16
NW = NC * NS
EPW = E // NW
CH = 125
NCHUNK = EPW // CH
TPT = 640
ZR = 80

_MESH = plsc.VectorSubcoreMesh(core_axis_name="c", subcore_axis_name="s")
_SC_PARAMS = pltpu.CompilerParams(use_tc_tiling_on_sc=False)


@functools.partial(
    pl.kernel,
    out_type=jax.ShapeDtypeStruct((NC * N, 8), jnp.float32),
    mesh=_MESH,
    compiler_params=_SC_PARAMS,
    scratch_types=[
        pltpu.VMEM((NCHUNK, CH), jnp.int32),
        pltpu.VMEM((CH, 8), jnp.float32),
        pltpu.VMEM((ZR, 8), jnp.float32),
        pltpu.VMEM_SHARED((N, 8), jnp.float32),
    ],
)
def _sc_degree(col_hbm, ones_hbm, zeros_hbm, out_hbm, cidx, ones_v, zbuf, acc):
    c = lax.axis_index("c")
    s = lax.axis_index("s")
    wid = s * NC + c
    nj = jnp.where(s == NS - 1, (N - (NS - 1) * TPT) // ZR, TPT // ZR)

    pltpu.sync_copy(col_hbm.at[wid], cidx)
    pltpu.sync_copy(ones_hbm, ones_v)
    pltpu.sync_copy(zeros_hbm, zbuf)

    def zero_body(j, _):
        r0 = pl.multiple_of(s * TPT + j * ZR, 8)
        pltpu.sync_copy(zbuf, acc.at[pl.ds(r0, ZR)])
        return _

    lax.fori_loop(0, nj, zero_body, None)
    plsc.subcore_barrier()

    def body(k, _):
        pltpu.sync_copy(ones_v, acc.at[cidx.at[k]], add=True)
        return _

    lax.fori_loop(0, NCHUNK, body, None)
    plsc.subcore_barrier()

    def out_body(j, _):
        r0 = pl.multiple_of(s * TPT + j * ZR, 8)
        pltpu.sync_copy(acc.at[pl.ds(r0, ZR)], zbuf)
        pltpu.sync_copy(zbuf, out_hbm.at[pl.ds(c * N + r0, ZR)])
        return _

    lax.fori_loop(0, nj, out_body, None)


@functools.partial(
    pl.kernel,
    out_type=jax.ShapeDtypeStruct((NC * N, H), jnp.float32),
    mesh=_MESH,
    compiler_params=_SC_PARAMS,
    scratch_types=[
        pltpu.VMEM((NCHUNK, CH), jnp.int32),
        pltpu.VMEM((NCHUNK, CH), jnp.int32),
        pltpu.VMEM((CH, H), jnp.float32),
        pltpu.VMEM((CH, H), jnp.float32),
        pltpu.VMEM((ZR, H), jnp.float32),
        pltpu.VMEM_SHARED((N, H), jnp.float32),
        pltpu.SemaphoreType.DMA,
        pltpu.SemaphoreType.DMA,
    ],
)
def _sc_aggregate(y_hbm, row_hbm, col_hbm, zeros_hbm, out_hbm,
                  ridx, cidx, buf0, buf1, zbuf, acc, g0, g1):
    c = lax.axis_index("c")
    s = lax.axis_index("s")
    wid = s * NC + c
    nj = jnp.where(s == NS - 1, (N - (NS - 1) * TPT) // ZR, TPT // ZR)

    pltpu.sync_copy(row_hbm.at[wid], ridx)
    pltpu.sync_copy(col_hbm.at[wid], cidx)
    pltpu.sync_copy(zeros_hbm, zbuf)

    def zero_body(j, _):
        r0 = pl.multiple_of(s * TPT + j * ZR, 8)
        pltpu.sync_copy(zbuf, acc.at[pl.ds(r0, ZR)])
        return _

    lax.fori_loop(0, nj, zero_body, None)
    plsc.subcore_barrier()

    pltpu.async_copy(y_hbm.at[ridx.at[0]], buf0, g0)
    pltpu.async_copy(y_hbm.at[ridx.at[1]], buf1, g1)

    def body(i, _):
        k = 2 * i
        pltpu.make_async_copy(y_hbm.at[ridx.at[k]], buf0, g0).wait()
        pltpu.sync_copy(buf0, acc.at[cidx.at[k]], add=True)

        @pl.when(k + 2 < NCHUNK)
        def _g0():
            pltpu.async_copy(y_hbm.at[ridx.at[k + 2]], buf0, g0)

        pltpu.make_async_copy(y_hbm.at[ridx.at[k + 1]], buf1, g1).wait()
        pltpu.sync_copy(buf1, acc.at[cidx.at[k + 1]], add=True)

        @pl.when(k + 3 < NCHUNK)
        def _g1():
            pltpu.async_copy(y_hbm.at[ridx.at[k + 3]], buf1, g1)

        return _

    lax.fori_loop(0, NCHUNK // 2, body, None)
    plsc.subcore_barrier()

    def out_body(j, _):
        r0 = pl.multiple_of(s * TPT + j * ZR, 8)
        pltpu.sync_copy(acc.at[pl.ds(r0, ZR)], zbuf)
        pltpu.sync_copy(zbuf, out_hbm.at[pl.ds(c * N + r0, ZR)])
        return _

    lax.fori_loop(0, nj, out_body, None)


def _tc_prep_body(deg_ref, x_ref, w0_ref, y_ref, xw_ref, dinv_ref):
    d8 = deg_ref[...]
    deg = d8[:N, 0:1] + d8[N:, 0:1] + 1.0
    dinv = lax.rsqrt(deg)
    xw = jnp.dot(x_ref[...], w0_ref[...], preferred_element_type=jnp.float32)
    y_ref[...] = xw * dinv
    xw_ref[...] = xw
    dinv_ref[...] = dinv


def _tc_prep(deg_parts, x, W0):
    return pl.pallas_call(
        _tc_prep_body,
        out_shape=[
            jax.ShapeDtypeStruct((N, H), jnp.float32),
            jax.ShapeDtypeStruct((N, H), jnp.float32),
            jax.ShapeDtypeStruct((N, 1), jnp.float32),
        ],
    )(deg_parts, x, W0)


def _layer_out(sp, xw, dinv, b, g, be):
    s = sp[:N] + sp[N:]
    o = dinv * s + (dinv * dinv) * xw + b
    mu = jnp.mean(o, axis=0, keepdims=True)
    var = jnp.mean((o - mu) * (o - mu), axis=0, keepdims=True)
    return jnp.maximum((o - mu) * lax.rsqrt(var + 1e-5) * g + be, 0.0)


def _tc_post_body(sp_ref, xw_ref, dinv_ref, b_ref, g_ref, be_ref, wn_ref,
                  yn_ref, xwn_ref):
    dinv = dinv_ref[...]
    h = _layer_out(sp_ref[...], xw_ref[...], dinv, b_ref[...], g_ref[...],
                   be_ref[...])
    xwn = jnp.dot(h, wn_ref[...], preferred_element_type=jnp.float32)
    yn_ref[...] = xwn * dinv
    xwn_ref[...] = xwn


def _tc_post(s_parts, xw, dinv, b, g, be, Wn):
    return pl.pallas_call(
        _tc_post_body,
        out_shape=[
            jax.ShapeDtypeStruct((N, H), jnp.float32),
            jax.ShapeDtypeStruct((N, H), jnp.float32),
        ],
    )(s_parts, xw, dinv, b.reshape(1, H), g.reshape(1, H), be.reshape(1, H),
      Wn)


def _tc_final_body(sp_ref, xw_ref, dinv_ref, b_ref, g_ref, be_ref,
                   wc1_ref, bc1_ref, wc2_ref, bc2_ref, out_ref):
    h = _layer_out(sp_ref[...], xw_ref[...], dinv_ref[...], b_ref[...],
                   g_ref[...], be_ref[...])
    hc = jnp.maximum(
        jnp.dot(h, wc1_ref[...], preferred_element_type=jnp.float32)
        + bc1_ref[...], 0.0)
    out_ref[...] = (
        jnp.dot(hc, wc2_ref[...], preferred_element_type=jnp.float32)
        + bc2_ref[...])


def _tc_final(s_parts, xw, dinv, b, g, be, Wc1, bc1, Wc2, bc2):
    return pl.pallas_call(
        _tc_final_body,
        out_shape=jax.ShapeDtypeStruct((N, C), jnp.float32),
    )(s_parts, xw, dinv, b.reshape(1, H), g.reshape(1, H), be.reshape(1, H),
      Wc1, bc1.reshape(1, H // 2), Wc2, bc2.reshape(1, C))


def kernel(x, edge_index, W0, b0, W1, b1, W2, b2, g0, be0, g1, be1, g2, be2,
           Wc1, bc1, Wc2, bc2):
    row3 = edge_index[0].reshape(NW, NCHUNK, CH)
    col3 = edge_index[1].reshape(NW, NCHUNK, CH)
    ones8 = jnp.ones((CH, 8), jnp.float32)
    zeros8 = jnp.zeros((ZR, 8), jnp.float32)
    zerosH = jnp.zeros((ZR, H), jnp.float32)

    deg_parts = _sc_degree(col3, ones8, zeros8)
    y0, xw0, dinv = _tc_prep(deg_parts, x, W0)
    s0 = _sc_aggregate(y0, row3, col3, zerosH)
    y1, xw1 = _tc_post(s0, xw0, dinv, b0, g0, be0, W1)
    s1 = _sc_aggregate(y1, row3, col3, zerosH)
    y2, xw2 = _tc_post(s1, xw1, dinv, b1, g1, be1, W2)
    s2 = _sc_aggregate(y2, row3, col3, zerosH)
    return _tc_final(s2, xw2, dinv, b2, g2, be2, Wc1, bc1, Wc2, bc2)


# speedup vs baseline: 36.3475x; 1.1490x over previous
import functools

import jax
import jax.numpy as jnp
from jax import lax
from jax.experimental import pallas as pl
from jax.experimental.pallas import tpu as pltpu
from jax.experimental.pallas import tpu_sc as plsc

N = 10000
E = 320000
F_IN = 128
H = 64
C = 10

NC = 2
NS = 16
NW = NC * NS
EPW = E // NW
CH = 125
NCHUNK = EPW // CH
TPT = 640
ZR = 80

_MESH = plsc.VectorSubcoreMesh(core_axis_name="c", subcore_axis_name="s")
_SC_PARAMS = pltpu.CompilerParams(use_tc_tiling_on_sc=False,
                                  disable_bounds_checks=True)


@functools.partial(
    pl.kernel,
    out_type=jax.ShapeDtypeStruct((NC * N, 8), jnp.float32),
    mesh=_MESH,
    compiler_params=_SC_PARAMS,
    scratch_types=[
        pltpu.VMEM((NCHUNK, CH), jnp.int32),
        pltpu.VMEM((CH, 8), jnp.float32),
        pltpu.VMEM((ZR, 8), jnp.float32),
        pltpu.VMEM_SHARED((N, 8), jnp.float32),
    ],
)
def _sc_degree(col_hbm, ones_hbm, zeros_hbm, out_hbm, cidx, ones_v, zbuf, acc):
    c = lax.axis_index("c")
    s = lax.axis_index("s")
    wid = s * NC + c
    nj = jnp.where(s == NS - 1, (N - (NS - 1) * TPT) // ZR, TPT // ZR)

    pltpu.sync_copy(col_hbm.at[wid], cidx)
    pltpu.sync_copy(ones_hbm, ones_v)
    pltpu.sync_copy(zeros_hbm, zbuf)

    def zero_body(j, _):
        r0 = pl.multiple_of(s * TPT + j * ZR, 8)
        pltpu.sync_copy(zbuf, acc.at[pl.ds(r0, ZR)])
        return _

    lax.fori_loop(0, nj, zero_body, None)
    plsc.subcore_barrier()

    def body(k, _):
        pltpu.sync_copy(ones_v, acc.at[cidx.at[k]], add=True)
        return _

    lax.fori_loop(0, NCHUNK, body, None)
    plsc.subcore_barrier()

    def out_body(j, _):
        r0 = pl.multiple_of(s * TPT + j * ZR, 8)
        pltpu.sync_copy(acc.at[pl.ds(r0, ZR)], zbuf)
        pltpu.sync_copy(zbuf, out_hbm.at[pl.ds(c * N + r0, ZR)])
        return _

    lax.fori_loop(0, nj, out_body, None)


@functools.partial(
    pl.kernel,
    out_type=jax.ShapeDtypeStruct((NC * N, H), jnp.float32),
    mesh=_MESH,
    compiler_params=_SC_PARAMS,
    scratch_types=[
        pltpu.VMEM((NCHUNK, CH), jnp.int32),
        pltpu.VMEM((NCHUNK, CH), jnp.int32),
        pltpu.VMEM((CH, H), jnp.float32),
        pltpu.VMEM((CH, H), jnp.float32),
        pltpu.VMEM((CH, H), jnp.float32),
        pltpu.VMEM((CH, H), jnp.float32),
        pltpu.VMEM((ZR, H), jnp.float32),
        pltpu.VMEM_SHARED((N, H), jnp.float32),
        pltpu.SemaphoreType.DMA,
        pltpu.SemaphoreType.DMA,
        pltpu.SemaphoreType.DMA,
        pltpu.SemaphoreType.DMA,
    ],
)
def _sc_aggregate(y_hbm, row_hbm, col_hbm, zeros_hbm, out_hbm,
                  ridx, cidx, buf0, buf1, buf2, buf3, zbuf, acc,
                  g0, g1, g2, g3):
    c = lax.axis_index("c")
    s = lax.axis_index("s")
    wid = s * NC + c
    nj = jnp.where(s == NS - 1, (N - (NS - 1) * TPT) // ZR, TPT // ZR)

    pltpu.sync_copy(row_hbm.at[wid], ridx)
    pltpu.sync_copy(col_hbm.at[wid], cidx)
    pltpu.sync_copy(zeros_hbm, zbuf)

    def zero_body(j, _):
        r0 = pl.multiple_of(s * TPT + j * ZR, 8)
        pltpu.sync_copy(zbuf, acc.at[pl.ds(r0, ZR)])
        return _

    lax.fori_loop(0, nj, zero_body, None)
    plsc.subcore_barrier()

    bufs = (buf0, buf1, buf2, buf3)
    sems = (g0, g1, g2, g3)
    nb = len(bufs)
    for b in range(nb):
        pltpu.async_copy(y_hbm.at[ridx.at[b]], bufs[b], sems[b])

    def body(i, _):
        k = nb * i
        for b in range(nb):
            pltpu.make_async_copy(y_hbm.at[ridx.at[k + b]], bufs[b],
                                  sems[b]).wait()
            pltpu.sync_copy(bufs[b], acc.at[cidx.at[k + b]], add=True)

            @pl.when(k + b + nb < NCHUNK)
            def _g():
                pltpu.async_copy(y_hbm.at[ridx.at[k + b + nb]], bufs[b],
                                 sems[b])

        return _

    lax.fori_loop(0, NCHUNK // nb, body, None)
    plsc.subcore_barrier()

    def out_body(j, _):
        r0 = pl.multiple_of(s * TPT + j * ZR, 8)
        pltpu.sync_copy(acc.at[pl.ds(r0, ZR)], zbuf)
        pltpu.sync_copy(zbuf, out_hbm.at[pl.ds(c * N + r0, ZR)])
        return _

    lax.fori_loop(0, nj, out_body, None)


def _tc_xw0_body(x_ref, w0_ref, xw_ref):
    xw_ref[...] = jnp.dot(x_ref[...], w0_ref[...],
                          preferred_element_type=jnp.float32)


def _tc_xw0(x, W0):
    return pl.pallas_call(
        _tc_xw0_body,
        out_shape=jax.ShapeDtypeStruct((N, H), jnp.float32),
    )(x, W0)


def _tc_prep_body(deg_ref, xw_ref, y_ref, dinv_ref):
    d8 = deg_ref[...]
    deg = d8[:N, 0:1] + d8[N:, 0:1] + 1.0
    dinv = lax.rsqrt(deg)
    y_ref[...] = xw_ref[...] * dinv
    dinv_ref[...] = dinv


def _tc_prep(deg_parts, xw):
    return pl.pallas_call(
        _tc_prep_body,
        out_shape=[
            jax.ShapeDtypeStruct((N, H), jnp.float32),
            jax.ShapeDtypeStruct((N, 1), jnp.float32),
        ],
    )(deg_parts, xw)


def _layer_out(sp, xw, dinv, b, g, be):
    s = sp[:N] + sp[N:]
    o = dinv * s + (dinv * dinv) * xw + b
    mu = jnp.mean(o, axis=0, keepdims=True)
    var = jnp.mean((o - mu) * (o - mu), axis=0, keepdims=True)
    return jnp.maximum((o - mu) * lax.rsqrt(var + 1e-5) * g + be, 0.0)


def _tc_post_body(sp_ref, xw_ref, dinv_ref, b_ref, g_ref, be_ref, wn_ref,
                  yn_ref, xwn_ref):
    dinv = dinv_ref[...]
    h = _layer_out(sp_ref[...], xw_ref[...], dinv, b_ref[...], g_ref[...],
                   be_ref[...])
    xwn = jnp.dot(h, wn_ref[...], preferred_element_type=jnp.float32)
    yn_ref[...] = xwn * dinv
    xwn_ref[...] = xwn


def _tc_post(s_parts, xw, dinv, b, g, be, Wn):
    return pl.pallas_call(
        _tc_post_body,
        out_shape=[
            jax.ShapeDtypeStruct((N, H), jnp.float32),
            jax.ShapeDtypeStruct((N, H), jnp.float32),
        ],
    )(s_parts, xw, dinv, b.reshape(1, H), g.reshape(1, H), be.reshape(1, H),
      Wn)


def _tc_final_body(sp_ref, xw_ref, dinv_ref, b_ref, g_ref, be_ref,
                   wc1_ref, bc1_ref, wc2_ref, bc2_ref, out_ref):
    h = _layer_out(sp_ref[...], xw_ref[...], dinv_ref[...], b_ref[...],
                   g_ref[...], be_ref[...])
    hc = jnp.maximum(
        jnp.dot(h, wc1_ref[...], preferred_element_type=jnp.float32)
        + bc1_ref[...], 0.0)
    out_ref[...] = (
        jnp.dot(hc, wc2_ref[...], preferred_element_type=jnp.float32)
        + bc2_ref[...])


def _tc_final(s_parts, xw, dinv, b, g, be, Wc1, bc1, Wc2, bc2):
    return pl.pallas_call(
        _tc_final_body,
        out_shape=jax.ShapeDtypeStruct((N, C), jnp.float32),
    )(s_parts, xw, dinv, b.reshape(1, H), g.reshape(1, H), be.reshape(1, H),
      Wc1, bc1.reshape(1, H // 2), Wc2, bc2.reshape(1, C))


def kernel(x, edge_index, W0, b0, W1, b1, W2, b2, g0, be0, g1, be1, g2, be2,
           Wc1, bc1, Wc2, bc2):
    row3 = edge_index[0].reshape(NW, NCHUNK, CH)
    col3 = edge_index[1].reshape(NW, NCHUNK, CH)
    ones8 = jnp.ones((CH, 8), jnp.float32)
    zeros8 = jnp.zeros((ZR, 8), jnp.float32)
    zerosH = jnp.zeros((ZR, H), jnp.float32)

    deg_parts = _sc_degree(col3, ones8, zeros8)
    xw0 = _tc_xw0(x, W0)
    y0, dinv = _tc_prep(deg_parts, xw0)
    s0 = _sc_aggregate(y0, row3, col3, zerosH)
    y1, xw1 = _tc_post(s0, xw0, dinv, b0, g0, be0, W1)
    s1 = _sc_aggregate(y1, row3, col3, zerosH)
    y2, xw2 = _tc_post(s1, xw1, dinv, b1, g1, be1, W2)
    s2 = _sc_aggregate(y2, row3, col3, zerosH)
    return _tc_final(s2, xw2, dinv, b2, g2, be2, Wc1, bc1, Wc2, bc2)
